# Initial kernel scaffold; baseline (speedup 1.0000x reference)
import functools

import jax
import jax.numpy as jnp
from jax import lax
from jax.experimental import pallas as pl
from jax.experimental.pallas import tpu as pltpu
from jax.experimental.pallas import tpu_sc as plsc

N = 100000
E = 3200000
G = 256

NC = 2
NS = 16

NP = 102400
NROWS = NP // 128
EP = 3211264
EROWS = EP // 128
GP = 512

SUP = 8
R = 2048
NBLK = NP // R

_f32 = jnp.float32


def _mesh():
    return plsc.VectorSubcoreMesh(core_axis_name="c", subcore_axis_name="s")


def _zero_acc(z_hbm, z_v, acc, s, rows_per_tile):
    pltpu.sync_copy(z_hbm, z_v)
    n = rows_per_tile // 128

    @pl.loop(0, n)
    def _(i):
        pltpu.sync_copy(z_v, acc.at[pl.ds(s * rows_per_tile + i * 128, 128)])


def _dump_acc(acc, out_hbm, c, s, rows_per_tile):
    pltpu.sync_copy(
        acc.at[pl.ds(s * rows_per_tile, rows_per_tile)],
        out_hbm.at[c, pl.ds(s * rows_per_tile, rows_per_tile)],
    )


def _deg_sc(dst2d, ones1, z1):
    @functools.partial(
        pl.kernel,
        mesh=_mesh(),
        out_type=jax.ShapeDtypeStruct((NC, NP, 1), _f32),
        scratch_types=[
            pltpu.VMEM((SUP, 128), jnp.int32),
            pltpu.VMEM((128, 1), _f32),
            pltpu.VMEM((128, 1), _f32),
            pltpu.VMEM_SHARED((NP, 1), _f32),
        ],
    )
    def k(dst_hbm, ones_hbm, z_hbm, out_hbm, dst_v, ones_v, z_v, acc):
        c = lax.axis_index("c")
        s = lax.axis_index("s")
        wid = c * NS + s
        pltpu.sync_copy(ones_hbm, ones_v)
        _zero_acc(z_hbm, z_v, acc, s, NP // NS)
        plsc.subcore_barrier()
        base = wid * (EROWS // (NC * NS))

        @pl.loop(0, EROWS // (NC * NS * SUP))
        def _(t):
            pltpu.sync_copy(dst_hbm.at[pl.ds(base + t * SUP, SUP)], dst_v)
            for j in range(SUP):
                pltpu.sync_copy(ones_v, acc.at[dst_v.at[j]], add=True)

        plsc.subcore_barrier()
        _dump_acc(acc, out_hbm, c, s, NP // NS)

    return k(dst2d, ones1, z1)


def _prop1_sc(src2d, dst2d, y1, z8):
    @functools.partial(
        pl.kernel,
        mesh=_mesh(),
        out_type=jax.ShapeDtypeStruct((NC, NP, 8), _f32),
        scratch_types=[
            pltpu.VMEM((SUP, 128), jnp.int32),
            pltpu.VMEM((SUP, 128), jnp.int32),
            pltpu.VMEM((SUP, 128, 8), _f32),
            pltpu.VMEM((128, 8), _f32),
            pltpu.VMEM_SHARED((NP, 8), _f32),
            pltpu.SemaphoreType.DMA,
        ],
    )
    def k(src_hbm, dst_hbm, tab_hbm, z_hbm, out_hbm,
          src_v, dst_v, rows_v, z_v, acc, sem):
        c = lax.axis_index("c")
        s = lax.axis_index("s")
        wid = c * NS + s
        _zero_acc(z_hbm, z_v, acc, s, NP // NS)
        plsc.subcore_barrier()
        base = wid * (EROWS // (NC * NS))

        @pl.loop(0, EROWS // (NC * NS * SUP))
        def _(t):
            r0 = base + t * SUP
            pltpu.sync_copy(src_hbm.at[pl.ds(r0, SUP)], src_v)
            pltpu.sync_copy(dst_hbm.at[pl.ds(r0, SUP)], dst_v)
            cps = [
                pltpu.async_copy(tab_hbm.at[src_v.at[j]], rows_v.at[j], sem)
                for j in range(SUP)
            ]
            for cp in cps:
                cp.wait()
            for j in range(SUP):
                pltpu.sync_copy(rows_v.at[j], acc.at[dst_v.at[j]], add=True)

        plsc.subcore_barrier()
        _dump_acc(acc, out_hbm, c, s, NP // NS)

    return k(src2d, dst2d, y1, z8)


def _prop2_sc(src2d, dst2d, y2a, y2b, z16):
    @functools.partial(
        pl.kernel,
        mesh=_mesh(),
        out_type=jax.ShapeDtypeStruct((NC, NP, 16), _f32),
        scratch_types=[
            pltpu.VMEM((SUP, 128), jnp.int32),
            pltpu.VMEM((SUP, 128), jnp.int32),
            pltpu.VMEM((SUP, 128, 16), _f32),
            pltpu.VMEM((128, 16), _f32),
            pltpu.VMEM_SHARED((NP, 16), _f32),
            pltpu.SemaphoreType.DMA,
        ],
    )
    def k(src_hbm, dst_hbm, taba_hbm, tabb_hbm, z_hbm, out_hbm,
          src_v, dst_v, rows_v, z_v, acc, sem):
        c = lax.axis_index("c")
        s = lax.axis_index("s")
        _zero_acc(z_hbm, z_v, acc, s, NP // NS)
        plsc.subcore_barrier()
        base = s * (EROWS // NS)

        def run(tab_hbm):
            @pl.loop(0, EROWS // (NS * SUP))
            def _(t):
                r0 = base + t * SUP
                pltpu.sync_copy(src_hbm.at[pl.ds(r0, SUP)], src_v)
                pltpu.sync_copy(dst_hbm.at[pl.ds(r0, SUP)], dst_v)
                cps = [
                    pltpu.async_copy(tab_hbm.at[src_v.at[j]], rows_v.at[j], sem)
                    for j in range(SUP)
                ]
                for cp in cps:
                    cp.wait()
                for j in range(SUP):
                    pltpu.sync_copy(rows_v.at[j], acc.at[dst_v.at[j]], add=True)

        @pl.when(c == 0)
        def _():
            run(taba_hbm)

        @pl.when(c == 1)
        def _():
            run(tabb_hbm)

        plsc.subcore_barrier()
        _dump_acc(acc, out_hbm, c, s, NP // NS)

    return k(src2d, dst2d, y2a, y2b, z16)


def _pool_sc(gp, batch2d, z8):
    @functools.partial(
        pl.kernel,
        mesh=_mesh(),
        out_type=jax.ShapeDtypeStruct((NC, GP, 8), _f32),
        scratch_types=[
            pltpu.VMEM((NROWS // (NC * NS), 128), jnp.int32),
            pltpu.VMEM((NP // (NC * NS), 8), _f32),
            pltpu.VMEM((128, 8), _f32),
            pltpu.VMEM_SHARED((GP, 8), _f32),
        ],
    )
    def k(g_hbm, b_hbm, z_hbm, out_hbm, b_v, g_v, z_v, acc):
        c = lax.axis_index("c")
        s = lax.axis_index("s")
        wid = c * NS + s
        rows_per_w = NP // (NC * NS)
        irows_per_w = NROWS // (NC * NS)
        pltpu.sync_copy(z_hbm, z_v)
        pltpu.sync_copy(
            z_v.at[pl.ds(0, GP // NS)], acc.at[pl.ds(s * (GP // NS), GP // NS)]
        )
        plsc.subcore_barrier()
        pltpu.sync_copy(g_hbm.at[pl.ds(wid * rows_per_w, rows_per_w)], g_v)
        pltpu.sync_copy(b_hbm.at[pl.ds(wid * irows_per_w, irows_per_w)], b_v)

        @pl.loop(0, irows_per_w)
        def _(j):
            pltpu.sync_copy(
                g_v.at[pl.ds(j * 128, 128)], acc.at[b_v.at[j]], add=True
            )

        plsc.subcore_barrier()
        pltpu.sync_copy(
            acc.at[pl.ds(s * (GP // NS), GP // NS)],
            out_hbm.at[c, pl.ds(s * (GP // NS), GP // NS)],
        )

    return k(gp, batch2d, z8)


def _stage_b_tc(d0, d1, xp):
    def body(d0_ref, d1_ref, x_ref, dinv_ref, y_ref):
        deg = d0_ref[...] + d1_ref[...] + 1.0
        dinv = lax.rsqrt(deg)
        dinv_ref[...] = dinv
        y_ref[...] = x_ref[...] * dinv

    col1 = pl.BlockSpec((R, 1), lambda i: (i, 0))
    col8 = pl.BlockSpec((R, 8), lambda i: (i, 0))
    return pl.pallas_call(
        body,
        grid=(NBLK,),
        in_specs=[col1, col1, col8],
        out_specs=[col1, col8],
        out_shape=[
            jax.ShapeDtypeStruct((NP, 1), _f32),
            jax.ShapeDtypeStruct((NP, 8), _f32),
        ],
    )(d0, d1, xp)


def _stage_d_tc(p1a, p1b, y1, dinv, w1, b1, w2):
    def body(pa_ref, pb_ref, y_ref, dinv_ref, w1_ref, b1_ref, w2_ref,
             ya_ref, yb_ref):
        t = (pa_ref[...] + pb_ref[...] + y_ref[...]) * dinv_ref[...]
        h1 = jnp.dot(t, w1_ref[...], preferred_element_type=_f32) + b1_ref[...]
        h1 = jnp.maximum(h1, 0.0)
        y2 = jnp.dot(h1, w2_ref[...], preferred_element_type=_f32)
        y2 = y2 * dinv_ref[...]
        ya_ref[...] = y2[:, :16]
        yb_ref[...] = y2[:, 16:]

    col1 = pl.BlockSpec((R, 1), lambda i: (i, 0))
    col8 = pl.BlockSpec((R, 8), lambda i: (i, 0))
    col16 = pl.BlockSpec((R, 16), lambda i: (i, 0))
    full = lambda shape: pl.BlockSpec(shape, lambda i: tuple(0 for _ in shape))
    return pl.pallas_call(
        body,
        grid=(NBLK,),
        in_specs=[col8, col8, col8, col1,
                  full((8, 64)), full((1, 64)), full((64, 32))],
        out_specs=[col16, col16],
        out_shape=[
            jax.ShapeDtypeStruct((NP, 16), _f32),
            jax.ShapeDtypeStruct((NP, 16), _f32),
        ],
    )(p1a, p1b, y1, dinv, w1, b1, w2)


def _stage_f_tc(p2a, p2b, y2a, y2b, dinv, b2a, b2b, w3a, w3b):
    def body(pa_ref, pb_ref, ya_ref, yb_ref, dinv_ref,
             b2a_ref, b2b_ref, w3a_ref, w3b_ref, g_ref):
        dinv = dinv_ref[...]
        h2a = jnp.maximum((pa_ref[...] + ya_ref[...]) * dinv + b2a_ref[...], 0.0)
        h2b = jnp.maximum((pb_ref[...] + yb_ref[...]) * dinv + b2b_ref[...], 0.0)
        g = (jnp.dot(h2a, w3a_ref[...], preferred_element_type=_f32)
             + jnp.dot(h2b, w3b_ref[...], preferred_element_type=_f32))
        col = lax.broadcasted_iota(jnp.int32, (R, 8), 1)
        g_ref[...] = g + (col == 5).astype(_f32)

    col1 = pl.BlockSpec((R, 1), lambda i: (i, 0))
    col8 = pl.BlockSpec((R, 8), lambda i: (i, 0))
    col16 = pl.BlockSpec((R, 16), lambda i: (i, 0))
    full = lambda shape: pl.BlockSpec(shape, lambda i: tuple(0 for _ in shape))
    return pl.pallas_call(
        body,
        grid=(NBLK,),
        in_specs=[col16, col16, col16, col16, col1,
                  full((1, 16)), full((1, 16)), full((16, 8)), full((16, 8))],
        out_specs=col8,
        out_shape=jax.ShapeDtypeStruct((NP, 8), _f32),
    )(p2a, p2b, y2a, y2b, dinv, b2a, b2b, w3a, w3b)


def kernel(x, edge_index, batch, W1, b1, W2, b2, W3, b3):
    src = edge_index[0].astype(jnp.int32)
    dst = edge_index[1].astype(jnp.int32)
    src_p = jnp.concatenate([src, jnp.zeros((EP - E,), jnp.int32)])
    dst_p = jnp.concatenate([dst, jnp.full((EP - E,), N, jnp.int32)])
    src2d = src_p.reshape(EROWS, 128)
    dst2d = dst_p.reshape(EROWS, 128)

    batch_p = jnp.concatenate(
        [batch.astype(jnp.int32), jnp.full((NP - N,), G, jnp.int32)]
    ).reshape(NROWS, 128)

    xp = jnp.pad(x, ((0, NP - N), (0, 1)))
    w1p = jnp.pad(W1, ((0, 1), (0, 0)))
    b1r = b1.reshape(1, 64)
    b2a = b2[:16].reshape(1, 16)
    b2b = b2[16:].reshape(1, 16)
    w3a = jnp.pad(W3[:16], ((0, 0), (0, 3)))
    w3b = jnp.pad(W3[16:], ((0, 0), (0, 3)))

    z1 = jnp.zeros((128, 1), _f32)
    z8 = jnp.zeros((128, 8), _f32)
    z16 = jnp.zeros((128, 16), _f32)
    ones1 = jnp.ones((128, 1), _f32)

    deg = _deg_sc(dst2d, ones1, z1)
    dinv, y1 = _stage_b_tc(deg[0], deg[1], xp)
    p1 = _prop1_sc(src2d, dst2d, y1, z8)
    y2a, y2b = _stage_d_tc(p1[0], p1[1], y1, dinv, w1p, b1r, W2)
    p2 = _prop2_sc(src2d, dst2d, y2a, y2b, z16)
    gp = _stage_f_tc(p2[0], p2[1], y2a, y2b, dinv, b2a, b2b, w3a, w3b)
    pools = _pool_sc(gp, batch_p, z8)

    pool = pools[0] + pools[1]
    sums = pool[:G, :5]
    cnt = pool[:G, 5:6]
    return (sums + cnt * b3[None, :]) / jnp.maximum(cnt, 1.0)


# speedup vs baseline: 37.9367x; 37.9367x over previous
import functools

import jax
import jax.numpy as jnp
from jax import lax
from jax.experimental import pallas as pl
from jax.experimental.pallas import tpu as pltpu
from jax.experimental.pallas import tpu_sc as plsc

N = 100000
E = 3200000
G = 256

NC = 2
NS = 16

NP = 102400
NROWS = NP // 128
EP = 3211264
EROWS = EP // 128
GP = 512

SUP = 8
R = 2048
NBLK = NP // R

_f32 = jnp.float32


def _mesh():
    return plsc.VectorSubcoreMesh(core_axis_name="c", subcore_axis_name="s")


_SC_PARAMS = pltpu.CompilerParams(use_tc_tiling_on_sc=False)


def _zero_acc(z_hbm, z_v, acc, s, rows_per_tile):
    pltpu.sync_copy(z_hbm, z_v)
    n = rows_per_tile // 128

    @pl.loop(0, n)
    def _(i):
        pltpu.sync_copy(z_v, acc.at[pl.ds(s * rows_per_tile + i * 128, 128)])


def _dump_acc(acc, out_hbm, c, s, rows_per_tile):
    pltpu.sync_copy(
        acc.at[pl.ds(s * rows_per_tile, rows_per_tile)],
        out_hbm.at[c, pl.ds(s * rows_per_tile, rows_per_tile)],
    )


def _deg_sc(dst2d, ones8, z8):
    @functools.partial(
        pl.kernel,
        mesh=_mesh(),
        compiler_params=_SC_PARAMS,
        out_type=jax.ShapeDtypeStruct((NC, NP, 8), _f32),
        scratch_types=[
            pltpu.VMEM((SUP, 128), jnp.int32),
            pltpu.VMEM((128, 8), _f32),
            pltpu.VMEM((128, 8), _f32),
            pltpu.VMEM_SHARED((NP, 8), _f32),
        ],
    )
    def k(dst_hbm, ones_hbm, z_hbm, out_hbm, dst_v, ones_v, z_v, acc):
        c = lax.axis_index("c")
        s = lax.axis_index("s")
        wid = c * NS + s
        pltpu.sync_copy(ones_hbm, ones_v)
        _zero_acc(z_hbm, z_v, acc, s, NP // NS)
        plsc.subcore_barrier()
        base = wid * (EROWS // (NC * NS))

        @pl.loop(0, EROWS // (NC * NS * SUP))
        def _(t):
            pltpu.sync_copy(dst_hbm.at[pl.ds(base + t * SUP, SUP)], dst_v)
            for j in range(SUP):
                pltpu.sync_copy(ones_v, acc.at[dst_v.at[j]], add=True)

        plsc.subcore_barrier()
        _dump_acc(acc, out_hbm, c, s, NP // NS)

    return k(dst2d, ones8, z8)


def _prop1_sc(src2d, dst2d, y1, z8):
    @functools.partial(
        pl.kernel,
        mesh=_mesh(),
        compiler_params=_SC_PARAMS,
        out_type=jax.ShapeDtypeStruct((NC, NP, 8), _f32),
        scratch_types=[
            pltpu.VMEM((SUP, 128), jnp.int32),
            pltpu.VMEM((SUP, 128), jnp.int32),
            pltpu.VMEM((SUP, 128, 8), _f32),
            pltpu.VMEM((128, 8), _f32),
            pltpu.VMEM_SHARED((NP, 8), _f32),
            pltpu.SemaphoreType.DMA,
        ],
    )
    def k(src_hbm, dst_hbm, tab_hbm, z_hbm, out_hbm,
          src_v, dst_v, rows_v, z_v, acc, sem):
        c = lax.axis_index("c")
        s = lax.axis_index("s")
        wid = c * NS + s
        _zero_acc(z_hbm, z_v, acc, s, NP // NS)
        plsc.subcore_barrier()
        base = wid * (EROWS // (NC * NS))

        @pl.loop(0, EROWS // (NC * NS * SUP))
        def _(t):
            r0 = base + t * SUP
            pltpu.sync_copy(src_hbm.at[pl.ds(r0, SUP)], src_v)
            pltpu.sync_copy(dst_hbm.at[pl.ds(r0, SUP)], dst_v)
            cps = [
                pltpu.async_copy(tab_hbm.at[src_v.at[j]], rows_v.at[j], sem)
                for j in range(SUP)
            ]
            for cp in cps:
                cp.wait()
            for j in range(SUP):
                pltpu.sync_copy(rows_v.at[j], acc.at[dst_v.at[j]], add=True)

        plsc.subcore_barrier()
        _dump_acc(acc, out_hbm, c, s, NP // NS)

    return k(src2d, dst2d, y1, z8)


def _prop2_sc(src2d, dst2d, y2a, y2b, z16):
    @functools.partial(
        pl.kernel,
        mesh=_mesh(),
        compiler_params=_SC_PARAMS,
        out_type=jax.ShapeDtypeStruct((NC, NP, 16), _f32),
        scratch_types=[
            pltpu.VMEM((SUP, 128), jnp.int32),
            pltpu.VMEM((SUP, 128), jnp.int32),
            pltpu.VMEM((SUP, 128, 16), _f32),
            pltpu.VMEM((128, 16), _f32),
            pltpu.VMEM_SHARED((NP, 16), _f32),
            pltpu.SemaphoreType.DMA,
        ],
    )
    def k(src_hbm, dst_hbm, taba_hbm, tabb_hbm, z_hbm, out_hbm,
          src_v, dst_v, rows_v, z_v, acc, sem):
        c = lax.axis_index("c")
        s = lax.axis_index("s")
        _zero_acc(z_hbm, z_v, acc, s, NP // NS)
        plsc.subcore_barrier()
        base = s * (EROWS // NS)

        def run(tab_hbm):
            @pl.loop(0, EROWS // (NS * SUP))
            def _(t):
                r0 = base + t * SUP
                pltpu.sync_copy(src_hbm.at[pl.ds(r0, SUP)], src_v)
                pltpu.sync_copy(dst_hbm.at[pl.ds(r0, SUP)], dst_v)
                cps = [
                    pltpu.async_copy(tab_hbm.at[src_v.at[j]], rows_v.at[j], sem)
                    for j in range(SUP)
                ]
                for cp in cps:
                    cp.wait()
                for j in range(SUP):
                    pltpu.sync_copy(rows_v.at[j], acc.at[dst_v.at[j]], add=True)

        @pl.when(c == 0)
        def _():
            run(taba_hbm)

        @pl.when(c == 1)
        def _():
            run(tabb_hbm)

        plsc.subcore_barrier()
        _dump_acc(acc, out_hbm, c, s, NP // NS)

    return k(src2d, dst2d, y2a, y2b, z16)


def _pool_sc(gp, batch2d, z8):
    @functools.partial(
        pl.kernel,
        mesh=_mesh(),
        compiler_params=_SC_PARAMS,
        out_type=jax.ShapeDtypeStruct((NC, GP, 8), _f32),
        scratch_types=[
            pltpu.VMEM((NROWS // (NC * NS), 128), jnp.int32),
            pltpu.VMEM((NP // (NC * NS), 8), _f32),
            pltpu.VMEM((128, 8), _f32),
            pltpu.VMEM_SHARED((GP, 8), _f32),
        ],
    )
    def k(g_hbm, b_hbm, z_hbm, out_hbm, b_v, g_v, z_v, acc):
        c = lax.axis_index("c")
        s = lax.axis_index("s")
        wid = c * NS + s
        rows_per_w = NP // (NC * NS)
        irows_per_w = NROWS // (NC * NS)
        pltpu.sync_copy(z_hbm, z_v)
        pltpu.sync_copy(
            z_v.at[pl.ds(0, GP // NS)], acc.at[pl.ds(s * (GP // NS), GP // NS)]
        )
        plsc.subcore_barrier()
        pltpu.sync_copy(g_hbm.at[pl.ds(wid * rows_per_w, rows_per_w)], g_v)
        pltpu.sync_copy(b_hbm.at[pl.ds(wid * irows_per_w, irows_per_w)], b_v)

        @pl.loop(0, irows_per_w)
        def _(j):
            pltpu.sync_copy(
                g_v.at[pl.ds(j * 128, 128)], acc.at[b_v.at[j]], add=True
            )

        plsc.subcore_barrier()
        pltpu.sync_copy(
            acc.at[pl.ds(s * (GP // NS), GP // NS)],
            out_hbm.at[c, pl.ds(s * (GP // NS), GP // NS)],
        )

    return k(gp, batch2d, z8)


def _stage_b_tc(d0, d1, xp):
    def body(d0_ref, d1_ref, x_ref, dinv_ref, y_ref):
        deg = d0_ref[...][:, :1] + d1_ref[...][:, :1] + 1.0
        dinv = lax.rsqrt(deg)
        dinv_ref[...] = dinv
        y_ref[...] = x_ref[...] * dinv

    col1 = pl.BlockSpec((R, 1), lambda i: (i, 0))
    col8 = pl.BlockSpec((R, 8), lambda i: (i, 0))
    return pl.pallas_call(
        body,
        grid=(NBLK,),
        in_specs=[col8, col8, col8],
        out_specs=[col1, col8],
        out_shape=[
            jax.ShapeDtypeStruct((NP, 1), _f32),
            jax.ShapeDtypeStruct((NP, 8), _f32),
        ],
    )(d0, d1, xp)


def _stage_d_tc(p1a, p1b, y1, dinv, w1, b1, w2):
    def body(pa_ref, pb_ref, y_ref, dinv_ref, w1_ref, b1_ref, w2_ref,
             ya_ref, yb_ref):
        t = (pa_ref[...] + pb_ref[...] + y_ref[...]) * dinv_ref[...]
        h1 = jnp.dot(t, w1_ref[...], preferred_element_type=_f32) + b1_ref[...]
        h1 = jnp.maximum(h1, 0.0)
        y2 = jnp.dot(h1, w2_ref[...], preferred_element_type=_f32)
        y2 = y2 * dinv_ref[...]
        ya_ref[...] = y2[:, :16]
        yb_ref[...] = y2[:, 16:]

    col1 = pl.BlockSpec((R, 1), lambda i: (i, 0))
    col8 = pl.BlockSpec((R, 8), lambda i: (i, 0))
    col16 = pl.BlockSpec((R, 16), lambda i: (i, 0))
    full = lambda shape: pl.BlockSpec(shape, lambda i: tuple(0 for _ in shape))
    return pl.pallas_call(
        body,
        grid=(NBLK,),
        in_specs=[col8, col8, col8, col1,
                  full((8, 64)), full((1, 64)), full((64, 32))],
        out_specs=[col16, col16],
        out_shape=[
            jax.ShapeDtypeStruct((NP, 16), _f32),
            jax.ShapeDtypeStruct((NP, 16), _f32),
        ],
    )(p1a, p1b, y1, dinv, w1, b1, w2)


def _stage_f_tc(p2a, p2b, y2a, y2b, dinv, b2a, b2b, w3a, w3b):
    def body(pa_ref, pb_ref, ya_ref, yb_ref, dinv_ref,
             b2a_ref, b2b_ref, w3a_ref, w3b_ref, g_ref):
        dinv = dinv_ref[...]
        h2a = jnp.maximum((pa_ref[...] + ya_ref[...]) * dinv + b2a_ref[...], 0.0)
        h2b = jnp.maximum((pb_ref[...] + yb_ref[...]) * dinv + b2b_ref[...], 0.0)
        g = (jnp.dot(h2a, w3a_ref[...], preferred_element_type=_f32)
             + jnp.dot(h2b, w3b_ref[...], preferred_element_type=_f32))
        col = lax.broadcasted_iota(jnp.int32, (R, 8), 1)
        g_ref[...] = g + (col == 5).astype(_f32)

    col1 = pl.BlockSpec((R, 1), lambda i: (i, 0))
    col8 = pl.BlockSpec((R, 8), lambda i: (i, 0))
    col16 = pl.BlockSpec((R, 16), lambda i: (i, 0))
    full = lambda shape: pl.BlockSpec(shape, lambda i: tuple(0 for _ in shape))
    return pl.pallas_call(
        body,
        grid=(NBLK,),
        in_specs=[col16, col16, col16, col16, col1,
                  full((1, 16)), full((1, 16)), full((16, 8)), full((16, 8))],
        out_specs=col8,
        out_shape=jax.ShapeDtypeStruct((NP, 8), _f32),
    )(p2a, p2b, y2a, y2b, dinv, b2a, b2b, w3a, w3b)


def kernel(x, edge_index, batch, W1, b1, W2, b2, W3, b3):
    src = edge_index[0].astype(jnp.int32)
    dst = edge_index[1].astype(jnp.int32)
    src_p = jnp.concatenate([src, jnp.zeros((EP - E,), jnp.int32)])
    dst_p = jnp.concatenate([dst, jnp.full((EP - E,), N, jnp.int32)])
    src2d = src_p.reshape(EROWS, 128)
    dst2d = dst_p.reshape(EROWS, 128)

    batch_p = jnp.concatenate(
        [batch.astype(jnp.int32), jnp.full((NP - N,), G, jnp.int32)]
    ).reshape(NROWS, 128)

    xp = jnp.pad(x, ((0, NP - N), (0, 1)))
    w1p = jnp.pad(W1, ((0, 1), (0, 0)))
    b1r = b1.reshape(1, 64)
    b2a = b2[:16].reshape(1, 16)
    b2b = b2[16:].reshape(1, 16)
    w3a = jnp.pad(W3[:16], ((0, 0), (0, 3)))
    w3b = jnp.pad(W3[16:], ((0, 0), (0, 3)))

    z8 = jnp.zeros((128, 8), _f32)
    z16 = jnp.zeros((128, 16), _f32)
    ones8 = jnp.ones((128, 8), _f32)

    deg = _deg_sc(dst2d, ones8, z8)
    dinv, y1 = _stage_b_tc(deg[0], deg[1], xp)
    p1 = _prop1_sc(src2d, dst2d, y1, z8)
    y2a, y2b = _stage_d_tc(p1[0], p1[1], y1, dinv, w1p, b1r, W2)
    p2 = _prop2_sc(src2d, dst2d, y2a, y2b, z16)
    gp = _stage_f_tc(p2[0], p2[1], y2a, y2b, dinv, b2a, b2b, w3a, w3b)
    pools = _pool_sc(gp, batch_p, z8)

    pool = pools[0] + pools[1]
    sums = pool[:G, :5]
    cnt = pool[:G, 5:6]
    return (sums + cnt * b3[None, :]) / jnp.maximum(cnt, 1.0)


# speedup vs baseline: 63.1590x; 1.6649x over previous
import functools

import jax
import jax.numpy as jnp
from jax import lax
from jax.experimental import pallas as pl
from jax.experimental.pallas import tpu as pltpu
from jax.experimental.pallas import tpu_sc as plsc

N = 100000
E = 3200000
G = 256

NC = 2
NS = 16

NP = 102400
NROWS = NP // 128
P8 = NP * 8 // 128
P16 = NP * 16 // 128
EP = 3211264
EROWS = EP // 128
GP = 512

SUP = 8

_f32 = jnp.float32


def _mesh():
    return plsc.VectorSubcoreMesh(core_axis_name="c", subcore_axis_name="s")


_SC_PARAMS = pltpu.CompilerParams(use_tc_tiling_on_sc=False)


def _zero_acc(z_hbm, z_v, acc, s, rows_per_tile):
    pltpu.sync_copy(z_hbm, z_v)
    n = rows_per_tile // 128

    @pl.loop(0, n)
    def _(i):
        pltpu.sync_copy(z_v, acc.at[pl.ds(s * rows_per_tile + i * 128, 128)])


def _dump_acc(acc, out_hbm, c, s, rows_per_tile):
    pltpu.sync_copy(
        acc.at[pl.ds(s * rows_per_tile, rows_per_tile)],
        out_hbm.at[c, pl.ds(s * rows_per_tile, rows_per_tile)],
    )


def _deg_sc(dst2d, ones8, z8):
    @functools.partial(
        pl.kernel,
        mesh=_mesh(),
        compiler_params=_SC_PARAMS,
        out_type=jax.ShapeDtypeStruct((NC, NP, 8), _f32),
        scratch_types=[
            pltpu.VMEM((SUP, 128), jnp.int32),
            pltpu.VMEM((128, 8), _f32),
            pltpu.VMEM((128, 8), _f32),
            pltpu.VMEM_SHARED((NP, 8), _f32),
            pltpu.SemaphoreType.DMA,
        ],
    )
    def k(dst_hbm, ones_hbm, z_hbm, out_hbm, dst_v, ones_v, z_v, acc, sem):
        c = lax.axis_index("c")
        s = lax.axis_index("s")
        wid = c * NS + s
        pltpu.sync_copy(ones_hbm, ones_v)
        _zero_acc(z_hbm, z_v, acc, s, NP // NS)
        plsc.subcore_barrier()
        base = wid * (EROWS // (NC * NS))

        @pl.loop(0, EROWS // (NC * NS * SUP))
        def _(t):
            pltpu.sync_copy(dst_hbm.at[pl.ds(base + t * SUP, SUP)], dst_v)
            ss = [
                pltpu.async_copy(ones_v, acc.at[dst_v.at[j]], sem, add=True)
                for j in range(SUP)
            ]
            for cp in ss:
                cp.wait()

        plsc.subcore_barrier()
        _dump_acc(acc, out_hbm, c, s, NP // NS)

    return k(dst2d, ones8, z8)


def _prop1_sc(src2d, dst2d, y1, z8):
    @functools.partial(
        pl.kernel,
        mesh=_mesh(),
        compiler_params=_SC_PARAMS,
        out_type=jax.ShapeDtypeStruct((NC, NP, 8), _f32),
        scratch_types=[
            pltpu.VMEM((SUP, 128), jnp.int32),
            pltpu.VMEM((SUP, 128), jnp.int32),
            pltpu.VMEM((SUP, 128, 8), _f32),
            pltpu.VMEM((128, 8), _f32),
            pltpu.VMEM_SHARED((NP, 8), _f32),
            pltpu.SemaphoreType.DMA,
            pltpu.SemaphoreType.DMA,
        ],
    )
    def k(src_hbm, dst_hbm, tab_hbm, z_hbm, out_hbm,
          src_v, dst_v, rows_v, z_v, acc, sem, ssem):
        c = lax.axis_index("c")
        s = lax.axis_index("s")
        wid = c * NS + s
        _zero_acc(z_hbm, z_v, acc, s, NP // NS)
        plsc.subcore_barrier()
        base = wid * (EROWS // (NC * NS))

        @pl.loop(0, EROWS // (NC * NS * SUP))
        def _(t):
            r0 = base + t * SUP
            pltpu.sync_copy(src_hbm.at[pl.ds(r0, SUP)], src_v)
            pltpu.sync_copy(dst_hbm.at[pl.ds(r0, SUP)], dst_v)
            gs = [
                pltpu.async_copy(tab_hbm.at[src_v.at[j]], rows_v.at[j], sem)
                for j in range(SUP)
            ]
            ss = []
            for j in range(SUP):
                gs[j].wait()
                ss.append(pltpu.async_copy(
                    rows_v.at[j], acc.at[dst_v.at[j]], ssem, add=True))
            for cp in ss:
                cp.wait()

        plsc.subcore_barrier()
        _dump_acc(acc, out_hbm, c, s, NP // NS)

    return k(src2d, dst2d, y1, z8)


def _prop2_sc(src2d, dst2d, y2a, y2b, z16):
    @functools.partial(
        pl.kernel,
        mesh=_mesh(),
        compiler_params=_SC_PARAMS,
        out_type=jax.ShapeDtypeStruct((NC, NP, 16), _f32),
        scratch_types=[
            pltpu.VMEM((SUP, 128), jnp.int32),
            pltpu.VMEM((SUP, 128), jnp.int32),
            pltpu.VMEM((SUP, 128, 16), _f32),
            pltpu.VMEM((128, 16), _f32),
            pltpu.VMEM_SHARED((NP, 16), _f32),
            pltpu.SemaphoreType.DMA,
            pltpu.SemaphoreType.DMA,
        ],
    )
    def k(src_hbm, dst_hbm, taba_hbm, tabb_hbm, z_hbm, out_hbm,
          src_v, dst_v, rows_v, z_v, acc, sem, ssem):
        c = lax.axis_index("c")
        s = lax.axis_index("s")
        _zero_acc(z_hbm, z_v, acc, s, NP // NS)
        plsc.subcore_barrier()
        base = s * (EROWS // NS)

        def run(tab_hbm):
            @pl.loop(0, EROWS // (NS * SUP))
            def _(t):
                r0 = base + t * SUP
                pltpu.sync_copy(src_hbm.at[pl.ds(r0, SUP)], src_v)
                pltpu.sync_copy(dst_hbm.at[pl.ds(r0, SUP)], dst_v)
                gs = [
                    pltpu.async_copy(tab_hbm.at[src_v.at[j]], rows_v.at[j], sem)
                    for j in range(SUP)
                ]
                ss = []
                for j in range(SUP):
                    gs[j].wait()
                    ss.append(pltpu.async_copy(
                        rows_v.at[j], acc.at[dst_v.at[j]], ssem, add=True))
                for cp in ss:
                    cp.wait()

        @pl.when(c == 0)
        def _():
            run(taba_hbm)

        @pl.when(c == 1)
        def _():
            run(tabb_hbm)

        plsc.subcore_barrier()
        _dump_acc(acc, out_hbm, c, s, NP // NS)

    return k(src2d, dst2d, y2a, y2b, z16)


def _pool_sc(g16, batch2d, z16):
    @functools.partial(
        pl.kernel,
        mesh=_mesh(),
        compiler_params=_SC_PARAMS,
        out_type=jax.ShapeDtypeStruct((NC, GP, 16), _f32),
        scratch_types=[
            pltpu.VMEM((NROWS // (NC * NS), 128), jnp.int32),
            pltpu.VMEM((NP // (NC * NS), 16), _f32),
            pltpu.VMEM((128, 16), _f32),
            pltpu.VMEM_SHARED((GP, 16), _f32),
        ],
    )
    def k(g_hbm, b_hbm, z_hbm, out_hbm, b_v, g_v, z_v, acc):
        c = lax.axis_index("c")
        s = lax.axis_index("s")
        wid = c * NS + s
        rows_per_w = NP // (NC * NS)
        irows_per_w = NROWS // (NC * NS)
        pltpu.sync_copy(z_hbm, z_v)
        pltpu.sync_copy(
            z_v.at[pl.ds(0, GP // NS)], acc.at[pl.ds(s * (GP // NS), GP // NS)]
        )
        plsc.subcore_barrier()
        pltpu.sync_copy(g_hbm.at[pl.ds(wid * rows_per_w, rows_per_w)], g_v)
        pltpu.sync_copy(b_hbm.at[pl.ds(wid * irows_per_w, irows_per_w)], b_v)

        @pl.loop(0, irows_per_w)
        def _(j):
            pltpu.sync_copy(
                g_v.at[pl.ds(j * 128, 128)], acc.at[b_v.at[j]], add=True
            )

        plsc.subcore_barrier()
        pltpu.sync_copy(
            acc.at[pl.ds(s * (GP // NS), GP // NS)],
            out_hbm.at[c, pl.ds(s * (GP // NS), GP // NS)],
        )

    return k(g16, batch2d, z16)


def _stage_b_tc(degp, xpk, rep16):
    def body(deg_ref, x_ref, rep_ref, y_ref, d8_ref, d16_ref):
        deg = deg_ref[0] + deg_ref[1] + 1.0
        dinv = lax.rsqrt(deg)
        d8_ref[...] = dinv
        y_ref[...] = x_ref[...] * dinv
        d16_ref[...] = jnp.dot(dinv, rep_ref[...],
                               preferred_element_type=_f32)

    nblk = P8 // 128
    blk = pl.BlockSpec((128, 128), lambda i: (i, 0))
    return pl.pallas_call(
        body,
        grid=(nblk,),
        in_specs=[
            pl.BlockSpec((NC, 128, 128), lambda i: (0, i, 0)),
            blk,
            pl.BlockSpec((128, 256), lambda i: (0, 0)),
        ],
        out_specs=[blk, blk, pl.BlockSpec((128, 256), lambda i: (i, 0))],
        out_shape=[
            jax.ShapeDtypeStruct((P8, 128), _f32),
            jax.ShapeDtypeStruct((P8, 128), _f32),
            jax.ShapeDtypeStruct((P8, 256), _f32),
        ],
    )(degp, xpk, rep16)


def _stage_d_tc(p1p, y1p, d8, d16, w1b, b1b, w2ab, w2bb):
    def body(p1_ref, y1_ref, d8_ref, d16_ref, w1_ref, b1_ref,
             w2a_ref, w2b_ref, ya_ref, yb_ref):
        t = (p1_ref[0] + p1_ref[1] + y1_ref[...]) * d8_ref[...]
        h1 = jnp.dot(t, w1_ref[...], preferred_element_type=_f32)
        h1 = jnp.maximum(h1 + b1_ref[...], 0.0)
        d16 = d16_ref[...]
        ya_ref[...] = jnp.dot(h1, w2a_ref[...],
                              preferred_element_type=_f32) * d16
        yb_ref[...] = jnp.dot(h1, w2b_ref[...],
                              preferred_element_type=_f32) * d16

    nblk = P8 // 128
    blk = pl.BlockSpec((128, 128), lambda i: (i, 0))
    blk2 = pl.BlockSpec((128, 256), lambda i: (i, 0))
    full = lambda shape: pl.BlockSpec(shape, lambda i: tuple(0 for _ in shape))
    return pl.pallas_call(
        body,
        grid=(nblk,),
        in_specs=[
            pl.BlockSpec((NC, 128, 128), lambda i: (0, i, 0)),
            blk, blk, blk2,
            full((128, 1024)), full((1, 1024)),
            full((1024, 256)), full((1024, 256)),
        ],
        out_specs=[blk2, blk2],
        out_shape=[
            jax.ShapeDtypeStruct((P8, 256), _f32),
            jax.ShapeDtypeStruct((P8, 256), _f32),
        ],
    )(p1p, y1p, d8, d16, w1b, b1b, w2ab, w2bb)


def _stage_f_tc(p2p, ya16, yb16, d16v, b2ab, b2bb, w3ab, w3bb):
    def body(p2_ref, ya_ref, yb_ref, d16_ref, b2a_ref, b2b_ref,
             w3a_ref, w3b_ref, g_ref):
        d16 = d16_ref[...]
        h2a = jnp.maximum((p2_ref[0] + ya_ref[...]) * d16 + b2a_ref[...], 0.0)
        h2b = jnp.maximum((p2_ref[1] + yb_ref[...]) * d16 + b2b_ref[...], 0.0)
        g = (jnp.dot(h2a, w3a_ref[...], preferred_element_type=_f32)
             + jnp.dot(h2b, w3b_ref[...], preferred_element_type=_f32))
        col = lax.broadcasted_iota(jnp.int32, (128, 128), 1)
        g_ref[...] = g + (col % 16 == 5).astype(_f32)

    nblk = P16 // 128
    blk = pl.BlockSpec((128, 128), lambda i: (i, 0))
    full = lambda shape: pl.BlockSpec(shape, lambda i: tuple(0 for _ in shape))
    return pl.pallas_call(
        body,
        grid=(nblk,),
        in_specs=[
            pl.BlockSpec((NC, 128, 128), lambda i: (0, i, 0)),
            blk, blk, blk,
            full((1, 128)), full((1, 128)),
            full((128, 128)), full((128, 128)),
        ],
        out_specs=blk,
        out_shape=jax.ShapeDtypeStruct((P16, 128), _f32),
    )(p2p, ya16, yb16, d16v, b2ab, b2bb, w3ab, w3bb)


def kernel(x, edge_index, batch, W1, b1, W2, b2, W3, b3):
    src = edge_index[0].astype(jnp.int32)
    dst = edge_index[1].astype(jnp.int32)
    src_p = jnp.concatenate([src, jnp.zeros((EP - E,), jnp.int32)])
    dst_p = jnp.concatenate([dst, jnp.full((EP - E,), N, jnp.int32)])
    src2d = src_p.reshape(EROWS, 128)
    dst2d = dst_p.reshape(EROWS, 128)

    batch_p = jnp.concatenate(
        [batch.astype(jnp.int32), jnp.full((NP - N,), G, jnp.int32)]
    ).reshape(NROWS, 128)

    xpk = jnp.pad(x, ((0, NP - N), (0, 1))).reshape(P8, 128)

    eye16 = jnp.eye(16, dtype=_f32)
    w1p = jnp.pad(W1, ((0, 1), (0, 0)))
    w1b = jnp.kron(eye16, w1p)
    b1b = jnp.tile(b1, 16).reshape(1, 1024)
    w2ab = jnp.kron(eye16, W2[:, :16])
    w2bb = jnp.kron(eye16, W2[:, 16:])
    b2ab = jnp.tile(b2[:16], 8).reshape(1, 128)
    b2bb = jnp.tile(b2[16:], 8).reshape(1, 128)
    eye8 = jnp.eye(8, dtype=_f32)
    w3a16 = jnp.pad(W3[:16], ((0, 0), (0, 11)))
    w3b16 = jnp.pad(W3[16:], ((0, 0), (0, 11)))
    w3ab = jnp.kron(eye8, w3a16)
    w3bb = jnp.kron(eye8, w3b16)
    e816 = jnp.zeros((8, 16), _f32).at[0, :].set(1.0)
    rep16 = jnp.kron(eye16, e816)

    z8 = jnp.zeros((128, 8), _f32)
    z16 = jnp.zeros((128, 16), _f32)
    ones8 = jnp.ones((128, 8), _f32)

    deg = _deg_sc(dst2d, ones8, z8)
    degp = deg.reshape(NC, P8, 128)
    y1p, d8, d16 = _stage_b_tc(degp, xpk, rep16)
    p1 = _prop1_sc(src2d, dst2d, y1p.reshape(NP, 8), z8)
    yab = _stage_d_tc(p1.reshape(NC, P8, 128), y1p, d8, d16,
                      w1b, b1b, w2ab, w2bb)
    p2 = _prop2_sc(src2d, dst2d,
                   yab[0].reshape(NP, 16), yab[1].reshape(NP, 16), z16)
    g16 = _stage_f_tc(p2.reshape(NC, P16, 128),
                      yab[0].reshape(P16, 128), yab[1].reshape(P16, 128),
                      d16.reshape(P16, 128), b2ab, b2bb, w3ab, w3bb)
    pools = _pool_sc(g16.reshape(NP, 16), batch_p, z16)

    pool = pools[0] + pools[1]
    sums = pool[:G, :5]
    cnt = pool[:G, 5:6]
    return (sums + cnt * b3[None, :]) / jnp.maximum(cnt, 1.0)
